# baseline (speedup 1.0000x reference)
import numpy as np

import jax
import jax.numpy as jnp
from jax.experimental import pallas as pl
from jax.experimental.pallas import tpu as pltpu

_D = 263
_B = 4
_T0 = 64
_WIDTH = 512
_CODE_DIM = 32
_NB_CODE = 256
_DEPTH = 3
_DOWN_T = 3
_DGR = 3

_MEAN_UPPER = np.asarray([0.1216, 0.2488, 0.2967, 0.5027, 0.4053, 0.41,
                          0.5703, 0.403, 0.4078, 0.1994, 0.1992, 0.0661,
                          0.0639], dtype=np.float32)
_STD_UPPER = np.asarray([0.0164, 0.0412, 0.0523, 0.0864, 0.0695, 0.0703,
                         0.1108, 0.0853, 0.0847, 0.1289, 0.1291, 0.2463,
                         0.2484], dtype=np.float32)
_SPINE_IDX = np.arange(0, 60)
_LA_IDX = np.arange(60, 108)
_RA_IDX = np.arange(101, 149)
_LL_IDX = np.arange(149, 208)
_RL_IDX = np.concatenate([np.arange(149, 153), np.arange(208, 263)])
_LOWER_MAP = np.array([0, 1, 2, 3])
_OVERLAP_LOWER_IDX = np.arange(149, 153)
_UPPER_Y_IDX = np.array([60 + 4 * i for i in range(13)])

_PARTS = ("left_arm", "right_arm", "right_leg", "left_leg", "spine")
_PART_IDX = {"left_arm": _LA_IDX, "right_arm": _RA_IDX, "right_leg": _RL_IDX,
             "left_leg": _LL_IDX, "spine": _SPINE_IDX}
_PART_DIM = {"left_arm": 48, "right_arm": 48, "right_leg": 59,
             "left_leg": 59, "spine": 60}


def _shift(x3, s):
    b, t, c = x3.shape
    if s == 0:
        return x3
    z = jnp.zeros((b, min(abs(s), t), c), dtype=x3.dtype)
    if abs(s) >= t:
        return z
    if s > 0:
        return jnp.concatenate([x3[:, s:, :], z], axis=1)
    return jnp.concatenate([z, x3[:, :s, :]], axis=1)


def _mm(a2, w_io):
    return jax.lax.dot_general(a2, w_io, (((1,), (0,)), ((), ())),
                               preferred_element_type=jnp.float32)


def _mmT(a2, w_oi):
    return jax.lax.dot_general(a2, w_oi, (((1,), (1,)), ((), ())),
                               preferred_element_type=jnp.float32)


def _taps(w2, k):
    o = w2.shape[0]
    wt = w2.T.reshape(w2.shape[1] // k, k, o)
    return [wt[:, j, :] for j in range(k)]


def _conv(x3, offsets, w2, bias):
    b, t, c = x3.shape
    taps = _taps(w2, len(offsets))
    acc = None
    for off, tap in zip(offsets, taps):
        if abs(off) >= t:
            continue
        y = _mm(_shift(x3, off).reshape(b * t, c), tap)
        acc = y if acc is None else acc + y
    return (acc + bias[None, :]).reshape(b, t, -1)


def _down_conv(x3, w2, bias):
    b, t, c = x3.shape
    to = t // 2
    taps = _taps(w2, 4)
    acc = None
    for k in range(4):
        xs = _shift(x3, k - 1).reshape(b, to, 2, c)[:, :, 0, :]
        y = _mm(xs.reshape(b * to, c), taps[k])
        acc = y if acc is None else acc + y
    return (acc + bias[None, :]).reshape(b, to, -1)


def _res_block(x3, w1, b1, w2, b2, d):
    h = jax.nn.relu(x3)
    h = _conv(h, (-d, 0, d), w1, b1)
    h = jax.nn.relu(h)
    b, t, c = h.shape
    y = _mmT(h.reshape(b * t, c), w2) + b2[None, :]
    return x3 + y.reshape(b, t, -1)


def _r2(w):
    return jnp.reshape(w, (w.shape[0], -1))


def _shift_upper_down(x):
    shift_y = x[:, :, 3:4]
    upper = (x[:, :, _UPPER_Y_IDX] - shift_y - _MEAN_UPPER) / _STD_UPPER
    return x.at[:, :, _UPPER_Y_IDX].set(upper)


def _shift_upper_up(x):
    upper = x[:, :, _UPPER_Y_IDX] * _STD_UPPER + _MEAN_UPPER
    x = x.at[:, :, _UPPER_Y_IDX].set(upper)
    shift_y = x[:, :, 3:4]
    return x.at[:, :, _UPPER_Y_IDX].add(shift_y)


def _merge(la, ra, rl, ll, sp):
    motion = jnp.zeros((_B, _T0, _D), dtype=la.dtype)
    motion = motion.at[:, :, _LA_IDX].set(la)
    motion = motion.at[:, :, _RA_IDX].set(ra)
    motion = motion.at[:, :, _RL_IDX].set(rl)
    motion = motion.at[:, :, _LL_IDX].set(ll)
    motion = motion.at[:, :, _SPINE_IDX].set(sp)
    return motion.at[:, :, _OVERLAP_LOWER_IDX].set(
        (ll[:, :, _LOWER_MAP] + rl[:, :, _LOWER_MAP]) / 2.0)


_N_AUTO = 14
_PER_PART = 42


def _eslab(p, s):
    return _N_AUTO + p * _PER_PART + s * 7


def _dslab(p, s):
    return _N_AUTO + p * _PER_PART + 21 + (s - 3) * 7


def _mega_kernel(*args):
    refs = args[:_N_AUTO + 5 * _PER_PART]
    (xp_ref, win_ref, bin_ref, benc_ref, wout_ref, bout_ref, cb_ref,
     dwin_ref, dbin_ref, bdec_ref, wmid_ref, bmid_ref, dwout_ref,
     dbout_ref) = refs[:_N_AUTO]
    y_ref, stats_ref = args[_N_AUTO + 5 * _PER_PART: _N_AUTO + 5 * _PER_PART + 2]
    (xbuf, qbuf, sa, sb, sc, sd, sema, semb, semc, semd) = \
        args[_N_AUTO + 5 * _PER_PART + 2:]

    pid = pl.program_id(0)

    def dmas(j, slot):
        p, s = divmod(j, 6)
        ops = []
        if s < 3:
            base = _eslab(p, s)
            ops.append(pltpu.make_async_copy(refs[base], sa.at[slot],
                                             sema.at[slot]))
            for jj in range(3):
                ops.append(pltpu.make_async_copy(refs[base + 1 + jj],
                                                 sb.at[slot, jj],
                                                 semb.at[slot, jj]))
                ops.append(pltpu.make_async_copy(refs[base + 4 + jj],
                                                 sc.at[slot, jj],
                                                 semc.at[slot, jj]))
        else:
            base = _dslab(p, s)
            for jj in range(3):
                ops.append(pltpu.make_async_copy(refs[base + jj],
                                                 sb.at[slot, jj],
                                                 semb.at[slot, jj]))
                ops.append(pltpu.make_async_copy(refs[base + 3 + jj],
                                                 sc.at[slot, jj],
                                                 semc.at[slot, jj]))
            ops.append(pltpu.make_async_copy(refs[base + 6], sd.at[slot],
                                             semd.at[slot]))
        return ops

    for j in range(30):
        pred = (pid == 0) if j == 0 else (pid == j - 1)

        def _start(j=j):
            for op in dmas(j, j % 2):
                op.start()
        pl.when(pred)(_start)

    for j in range(30):
        def _wait(j=j):
            for op in dmas(j, j % 2):
                op.wait()
        pl.when(pid == j)(_wait)

    slot = jax.lax.rem(pid, 2)

    def enc_stage(h):
        h = _down_conv(h, sa[slot], benc_ref[0, 0, 0])
        for j in range(_DEPTH):
            h = _res_block(h, sb[slot, j], benc_ref[0, 0, 1 + 2 * j],
                           sc[slot, j], benc_ref[0, 0, 2 + 2 * j], _DGR ** j)
        return h

    def dec_stage(h):
        for j in range(_DEPTH):
            h = _res_block(h, sb[slot, j], bdec_ref[0, 0, 2 * j],
                           sc[slot, j], bdec_ref[0, 0, 2 * j + 1],
                           _DGR ** (_DEPTH - 1 - j))
        b, t, c = h.shape
        h = jnp.broadcast_to(h[:, :, None, :], (b, t, 2, c)).reshape(
            b, 2 * t, c)
        return _conv(h, (-1, 0, 1), sd[slot], bdec_ref[0, 0, 6])

    sidx = jax.lax.rem(pid, 6)

    @pl.when(sidx == 0)
    def _s0():
        h = jax.nn.relu(_conv(xp_ref[0], (-1, 0, 1), win_ref[0],
                              bin_ref[0, 0]))
        xbuf[:, :32, :] = enc_stage(h)

    @pl.when(sidx == 1)
    def _s1():
        xbuf[:, :16, :] = enc_stage(xbuf[:, :32, :])

    @pl.when(sidx == 2)
    def _s2():
        h = enc_stage(xbuf[:, :16, :])
        e = _conv(h, (-1, 0, 1), wout_ref[0], bout_ref[0, 0])
        n = _B * 8
        xf = e.reshape(n, _CODE_DIM)
        cb = cb_ref[0]
        dist = (jnp.sum(xf * xf, axis=1, keepdims=True)
                - 2.0 * _mmT(xf, cb)
                + jnp.sum(cb * cb, axis=1)[None, :])
        dmin = jnp.min(dist, axis=1, keepdims=True)
        lane = jax.lax.broadcasted_iota(jnp.int32, (n, _NB_CODE), 1)
        idx = jnp.min(jnp.where(dist <= dmin, lane, _NB_CODE), axis=1)
        onehot = (lane == idx[:, None]).astype(jnp.float32)
        xd = _mm(onehot, cb)
        loss = jnp.mean((xf - xd) ** 2)
        pr = jnp.mean(onehot, axis=0)
        perp = jnp.exp(-jnp.sum(pr * jnp.log(pr + 1e-10)))
        qbuf[...] = xd.reshape(_B, 8, _CODE_DIM)
        row = jax.lax.broadcasted_iota(jnp.int32, (8, 128), 0)
        stats_ref[0] = jnp.where(row == 0, loss,
                                 jnp.where(row == 1, perp, 0.0))

    @pl.when(sidx == 3)
    def _s3():
        h = jax.nn.relu(_conv(qbuf[...], (-1, 0, 1), dwin_ref[0],
                              dbin_ref[0, 0]))
        xbuf[:, :16, :] = dec_stage(h)

    @pl.when(sidx == 4)
    def _s4():
        h = dec_stage(xbuf[:, :16, :])
        xbuf[:, :32, :] = h

    @pl.when(sidx == 5)
    def _s5():
        h = dec_stage(xbuf[:, :32, :])
        h = jax.nn.relu(_conv(h, (-1, 0, 1), wmid_ref[0], bmid_ref[0, 0]))
        y_ref[0] = _conv(h, (-1, 0, 1), dwout_ref[0], dbout_ref[0, 0])


def _pad_ax(a, ax, n):
    pad = n - a.shape[ax]
    if pad == 0:
        return a
    cfg = [(0, 0)] * a.ndim
    cfg[ax] = (0, pad)
    return jnp.pad(a, cfg)


def kernel(x, params):
    x = x.astype(jnp.float32)
    xs = _shift_upper_down(x)
    f32 = jnp.float32

    xp5 = jnp.stack([_pad_ax(xs[:, :, _PART_IDX[n]], 2, 64) for n in _PARTS])
    win5, bin5, benc5, wout5, bout5, cb5 = [], [], [], [], [], []
    dwin5, dbin5, bdec5, wmid5, bmid5, dwout5, dbout5 = ([] for _ in range(7))
    big = []
    z = jnp.zeros((_WIDTH,), f32)
    for n in _PARTS:
        enc, dec = params["enc"][n], params["dec"][n]
        win5.append(_pad_ax(enc["w_in"], 1, 64).reshape(_WIDTH, 192))
        bin5.append(enc["b_in"][None])
        eb = []
        for blk in enc["down"]:
            rows = [blk["b"]]
            for rb in blk["res"]:
                rows += [rb["b1"], rb["b2"]]
            rows.append(z)
            eb.append(jnp.stack(rows))
            big.append(_r2(blk["w"]))
            for rb in blk["res"]:
                big.append(_r2(rb["w1"]))
            for rb in blk["res"]:
                big.append(_r2(rb["w2"]))
        benc5.append(jnp.stack(eb))
        wout5.append(_r2(enc["w_out"]))
        bout5.append(enc["b_out"][None])
        cb5.append(params["cb"][n])
        dwin5.append(_r2(dec["w_in"]))
        dbin5.append(dec["b_in"][None])
        db_ = []
        for blk in dec["up"]:
            rows = []
            for rb in blk["res"]:
                rows += [rb["b1"], rb["b2"]]
            rows += [blk["b"], z]
            db_.append(jnp.stack(rows))
            for rb in blk["res"]:
                big.append(_r2(rb["w1"]))
            for rb in blk["res"]:
                big.append(_r2(rb["w2"]))
            big.append(_r2(blk["w"]))
        bdec5.append(jnp.stack(db_))
        wmid5.append(_r2(dec["w_mid"]))
        bmid5.append(dec["b_mid"][None])
        dwout5.append(_pad_ax(dec["w_out"].reshape(-1, 1536), 0, 64))
        dbout5.append(_pad_ax(dec["b_out"], 0, 64)[None])

    autos = [xp5] + [jnp.stack(v) for v in (
        win5, bin5, benc5, wout5, bout5, cb5, dwin5, dbin5, bdec5,
        wmid5, bmid5, dwout5, dbout5)]

    def part_idx(r):
        return lambda i: (i // 6,) + (0,) * r

    auto_specs = [
        pl.BlockSpec((1, _B, _T0, 64), part_idx(3)),
        pl.BlockSpec((1, _WIDTH, 192), part_idx(2)),
        pl.BlockSpec((1, 1, _WIDTH), part_idx(2)),
        pl.BlockSpec((1, 1, 8, _WIDTH),
                     lambda i: (i // 6, jnp.minimum(i % 6, 2), 0, 0)),
        pl.BlockSpec((1, _CODE_DIM, 1536), part_idx(2)),
        pl.BlockSpec((1, 1, _CODE_DIM), part_idx(2)),
        pl.BlockSpec((1, _NB_CODE, _CODE_DIM), part_idx(2)),
        pl.BlockSpec((1, _WIDTH, 96), part_idx(2)),
        pl.BlockSpec((1, 1, _WIDTH), part_idx(2)),
        pl.BlockSpec((1, 1, 8, _WIDTH),
                     lambda i: (i // 6, jnp.clip(i % 6 - 3, 0, 2), 0, 0)),
        pl.BlockSpec((1, _WIDTH, 1536), part_idx(2)),
        pl.BlockSpec((1, 1, _WIDTH), part_idx(2)),
        pl.BlockSpec((1, 64, 1536), part_idx(2)),
        pl.BlockSpec((1, 1, 64), part_idx(2)),
    ]
    any_spec = pl.BlockSpec(memory_space=pl.ANY)

    y5, stats5 = pl.pallas_call(
        _mega_kernel,
        grid=(30,),
        in_specs=auto_specs + [any_spec] * (5 * _PER_PART),
        out_specs=[
            pl.BlockSpec((1, _B, _T0, 64), part_idx(3)),
            pl.BlockSpec((1, 8, 128), part_idx(2)),
        ],
        out_shape=[
            jax.ShapeDtypeStruct((5, _B, _T0, 64), f32),
            jax.ShapeDtypeStruct((5, 8, 128), f32),
        ],
        scratch_shapes=[
            pltpu.VMEM((_B, _T0, _WIDTH), f32),
            pltpu.VMEM((_B, 8, _CODE_DIM), f32),
            pltpu.VMEM((2, _WIDTH, 2048), f32),
            pltpu.VMEM((2, 3, _WIDTH, 1536), f32),
            pltpu.VMEM((2, 3, _WIDTH, _WIDTH), f32),
            pltpu.VMEM((2, _WIDTH, 1536), f32),
            pltpu.SemaphoreType.DMA((2,)),
            pltpu.SemaphoreType.DMA((2, 3)),
            pltpu.SemaphoreType.DMA((2, 3)),
            pltpu.SemaphoreType.DMA((2,)),
        ],
    )(*autos, *big)

    ys = [y5[i][:, :, :_PART_DIM[n]] for i, n in enumerate(_PARTS)]
    motion = _shift_upper_up(_merge(ys[0], ys[1], ys[2], ys[3], ys[4]))
    loss = jnp.sum(stats5[:, 0, 0])
    perplexity = stats5[4, 1, 0]
    return motion, loss, perplexity


# speedup vs baseline: 2.8899x; 2.8899x over previous
import numpy as np

import jax
import jax.numpy as jnp
from jax.experimental import pallas as pl
from jax.experimental.pallas import tpu as pltpu

_D = 263
_B = 4
_T0 = 64
_WIDTH = 512
_CODE_DIM = 32
_NB_CODE = 256
_DEPTH = 3
_DOWN_T = 3
_DGR = 3

_MEAN_UPPER = np.asarray([0.1216, 0.2488, 0.2967, 0.5027, 0.4053, 0.41,
                          0.5703, 0.403, 0.4078, 0.1994, 0.1992, 0.0661,
                          0.0639], dtype=np.float32)
_STD_UPPER = np.asarray([0.0164, 0.0412, 0.0523, 0.0864, 0.0695, 0.0703,
                         0.1108, 0.0853, 0.0847, 0.1289, 0.1291, 0.2463,
                         0.2484], dtype=np.float32)
_SPINE_IDX = np.arange(0, 60)
_LA_IDX = np.arange(60, 108)
_RA_IDX = np.arange(101, 149)
_LL_IDX = np.arange(149, 208)
_RL_IDX = np.concatenate([np.arange(149, 153), np.arange(208, 263)])
_LOWER_MAP = np.array([0, 1, 2, 3])
_OVERLAP_LOWER_IDX = np.arange(149, 153)
_UPPER_Y_IDX = np.array([60 + 4 * i for i in range(13)])

_PARTS = ("left_arm", "right_arm", "right_leg", "left_leg", "spine")
_PART_IDX = {"left_arm": _LA_IDX, "right_arm": _RA_IDX, "right_leg": _RL_IDX,
             "left_leg": _LL_IDX, "spine": _SPINE_IDX}
_PART_DIM = {"left_arm": 48, "right_arm": 48, "right_leg": 59,
             "left_leg": 59, "spine": 60}


def _shift(x3, s):
    b, t, c = x3.shape
    if s == 0:
        return x3
    z = jnp.zeros((b, min(abs(s), t), c), dtype=x3.dtype)
    if abs(s) >= t:
        return z
    if s > 0:
        return jnp.concatenate([x3[:, s:, :], z], axis=1)
    return jnp.concatenate([z, x3[:, :s, :]], axis=1)


def _mm(a2, w_io):
    return jax.lax.dot_general(a2, w_io, (((1,), (0,)), ((), ())),
                               preferred_element_type=jnp.float32)


def _mmT(a2, w_oi):
    return jax.lax.dot_general(a2, w_oi, (((1,), (1,)), ((), ())),
                               preferred_element_type=jnp.float32)


def _conv_taps(x3, shifts, taps, bias):
    b, t, c = x3.shape
    acc = None
    for s, w_oi in zip(shifts, taps):
        if abs(s) >= t:
            continue
        y = _mmT(_shift(x3, s).reshape(b * t, c), w_oi)
        acc = y if acc is None else acc + y
    return (acc + bias[None, :]).reshape(b, t, -1)


def _down_conv(x3, taps, bias):
    b, t, c = x3.shape
    to = t // 2
    acc = None
    for k in range(4):
        xs = _shift(x3, k - 1).reshape(b, to, 2, c)[:, :, 0, :]
        y = _mmT(xs.reshape(b * to, c), taps[k])
        acc = y if acc is None else acc + y
    return (acc + bias[None, :]).reshape(b, to, -1)


def _res_block(x3, w1taps, b1, w2, b2, d):
    h = jax.nn.relu(x3)
    h = _conv_taps(h, (-d, 0, d), w1taps, b1)
    h = jax.nn.relu(h)
    b, t, c = h.shape
    y = _mmT(h.reshape(b * t, c), w2) + b2[None, :]
    return x3 + y.reshape(b, t, -1)


def _enc_level(xv, wlev_ref, blev_ref):
    y = _down_conv(xv, [wlev_ref[0, 0, k] for k in range(4)],
                   blev_ref[0, 0, 0])
    for j in range(_DEPTH):
        base = 4 + 4 * j
        y = _res_block(y, [wlev_ref[0, 0, base + k] for k in range(3)],
                       blev_ref[0, 0, 1 + 2 * j],
                       wlev_ref[0, 0, base + 3],
                       blev_ref[0, 0, 2 + 2 * j], _DGR ** j)
    return y


def _enc_kernel(xin_ref, win_ref, bin_ref, wlev_ref, blev_ref, wout_ref,
                bout_ref, cb_ref, q_ref, stats_ref, xs_ref):
    lvl = pl.program_id(1)

    @pl.when(lvl == 0)
    def _l0():
        h = jax.nn.relu(_conv_taps(xin_ref[0], (-1, 0, 1),
                                   [win_ref[0, k] for k in range(3)],
                                   bin_ref[0, 0]))
        xs_ref[:, :32, :] = _enc_level(h, wlev_ref, blev_ref)

    @pl.when(lvl == 1)
    def _l1():
        xs_ref[:, :16, :] = _enc_level(xs_ref[:, :32, :], wlev_ref, blev_ref)

    @pl.when(lvl == 2)
    def _l2():
        y = _enc_level(xs_ref[:, :16, :], wlev_ref, blev_ref)
        e = _conv_taps(y, (-1, 0, 1), [wout_ref[0, k] for k in range(3)],
                       bout_ref[0, 0])
        n = _B * 8
        xf = e.reshape(n, _CODE_DIM)
        cb = cb_ref[0]
        dist = (jnp.sum(xf * xf, axis=1, keepdims=True)
                - 2.0 * _mmT(xf, cb)
                + jnp.sum(cb * cb, axis=1)[None, :])
        dmin = jnp.min(dist, axis=1, keepdims=True)
        lane = jax.lax.broadcasted_iota(jnp.int32, (n, _NB_CODE), 1)
        idx = jnp.min(jnp.where(dist <= dmin, lane, _NB_CODE), axis=1)
        onehot = (lane == idx[:, None]).astype(jnp.float32)
        xd = _mm(onehot, cb)
        loss = jnp.mean((xf - xd) ** 2)
        pr = jnp.mean(onehot, axis=0)
        perp = jnp.exp(-jnp.sum(pr * jnp.log(pr + 1e-10)))
        q_ref[0] = xd.reshape(_B, 8, _CODE_DIM)
        row = jax.lax.broadcasted_iota(jnp.int32, (8, 128), 0)
        stats_ref[0] = jnp.where(row == 0, loss,
                                 jnp.where(row == 1, perp, 0.0))


def _dec_level(xv, wlev_ref, blev_ref):
    for j in range(_DEPTH):
        base = 4 * j
        xv = _res_block(xv, [wlev_ref[0, 0, base + k] for k in range(3)],
                        blev_ref[0, 0, 2 * j],
                        wlev_ref[0, 0, base + 3],
                        blev_ref[0, 0, 2 * j + 1], _DGR ** (_DEPTH - 1 - j))
    b, t, c = xv.shape
    xv = jnp.broadcast_to(xv[:, :, None, :], (b, t, 2, c)).reshape(b, 2 * t, c)
    return _conv_taps(xv, (-1, 0, 1),
                      [wlev_ref[0, 0, 12 + k] for k in range(3)],
                      blev_ref[0, 0, 6])


def _dec_kernel(q_ref, win_ref, bin_ref, wlev_ref, blev_ref, wmid_ref,
                bmid_ref, wout_ref, bout_ref, y_ref, xs_ref):
    lvl = pl.program_id(1)

    @pl.when(lvl == 0)
    def _l0():
        h = jax.nn.relu(_conv_taps(q_ref[0], (-1, 0, 1),
                                   [win_ref[0, k] for k in range(3)],
                                   bin_ref[0, 0]))
        xs_ref[:, :16, :] = _dec_level(h, wlev_ref, blev_ref)

    @pl.when(lvl == 1)
    def _l1():
        xs_ref[:, :32, :] = _dec_level(xs_ref[:, :16, :], wlev_ref, blev_ref)

    @pl.when(lvl == 2)
    def _l2():
        h = _dec_level(xs_ref[:, :32, :], wlev_ref, blev_ref)
        h = jax.nn.relu(_conv_taps(h, (-1, 0, 1),
                                   [wmid_ref[0, k] for k in range(3)],
                                   bmid_ref[0, 0]))
        y_ref[0] = _conv_taps(h, (-1, 0, 1),
                              [wout_ref[0, k] for k in range(3)],
                              bout_ref[0, 0])


def _shift_upper_down(x):
    shift_y = x[:, :, 3:4]
    upper = (x[:, :, _UPPER_Y_IDX] - shift_y - _MEAN_UPPER) / _STD_UPPER
    return x.at[:, :, _UPPER_Y_IDX].set(upper)


def _shift_upper_up(x):
    upper = x[:, :, _UPPER_Y_IDX] * _STD_UPPER + _MEAN_UPPER
    x = x.at[:, :, _UPPER_Y_IDX].set(upper)
    shift_y = x[:, :, 3:4]
    return x.at[:, :, _UPPER_Y_IDX].add(shift_y)


def _merge(la, ra, rl, ll, sp):
    motion = jnp.zeros((_B, _T0, _D), dtype=la.dtype)
    motion = motion.at[:, :, _LA_IDX].set(la)
    motion = motion.at[:, :, _RA_IDX].set(ra)
    motion = motion.at[:, :, _RL_IDX].set(rl)
    motion = motion.at[:, :, _LL_IDX].set(ll)
    motion = motion.at[:, :, _SPINE_IDX].set(sp)
    return motion.at[:, :, _OVERLAP_LOWER_IDX].set(
        (ll[:, :, _LOWER_MAP] + rl[:, :, _LOWER_MAP]) / 2.0)


def _mv(w):
    return jnp.moveaxis(w, 2, 0)


def _pad_ax(a, ax, n):
    pad = n - a.shape[ax]
    if pad == 0:
        return a
    cfg = [(0, 0)] * a.ndim
    cfg[ax] = (0, pad)
    return jnp.pad(a, cfg)


def kernel(x, params):
    x = x.astype(jnp.float32)
    xs = _shift_upper_down(x)
    f32 = jnp.float32

    xp5 = jnp.stack([_pad_ax(xs[:, :, _PART_IDX[n]], 2, 64) for n in _PARTS])
    z = jnp.zeros((_WIDTH,), f32)
    win, bin_, wlev, blev, wout, bout, cb = [], [], [], [], [], [], []
    dwin, dbin, dwlev, dblev, wmid, bmid, dwout, dbout = \
        ([] for _ in range(8))
    for n in _PARTS:
        enc = params["enc"][n]
        dec = params["dec"][n]
        win.append(_pad_ax(_mv(enc["w_in"]), 2, 64))
        bin_.append(enc["b_in"][None])
        lw, lb = [], []
        for blk in enc["down"]:
            taps = [_mv(blk["w"])]
            rows = [blk["b"]]
            for rb in blk["res"]:
                taps += [_mv(rb["w1"]), _mv(rb["w2"])]
                rows += [rb["b1"], rb["b2"]]
            rows.append(z)
            lw.append(jnp.concatenate(taps))
            lb.append(jnp.stack(rows))
        wlev.append(jnp.stack(lw))
        blev.append(jnp.stack(lb))
        wout.append(_mv(enc["w_out"]))
        bout.append(enc["b_out"][None])
        cb.append(params["cb"][n])
        dwin.append(_mv(dec["w_in"]))
        dbin.append(dec["b_in"][None])
        lw, lb = [], []
        for blk in dec["up"]:
            taps, rows = [], []
            for rb in blk["res"]:
                taps += [_mv(rb["w1"]), _mv(rb["w2"])]
                rows += [rb["b1"], rb["b2"]]
            taps.append(_mv(blk["w"]))
            rows += [blk["b"], z]
            lw.append(jnp.concatenate(taps))
            lb.append(jnp.stack(rows))
        dwlev.append(jnp.stack(lw))
        dblev.append(jnp.stack(lb))
        wmid.append(_mv(dec["w_mid"]))
        bmid.append(dec["b_mid"][None])
        dwout.append(_pad_ax(_mv(dec["w_out"]), 1, 64))
        dbout.append(_pad_ax(dec["b_out"], 0, 64)[None])

    grid = (5, _DOWN_T)
    W = _WIDTH

    enc_in = [xp5] + [jnp.stack(v) for v in
                      (win, bin_, wlev, blev, wout, bout, cb)]
    q5, stats5 = pl.pallas_call(
        _enc_kernel,
        grid=grid,
        in_specs=[
            pl.BlockSpec((1, _B, _T0, 64), lambda p, l: (p, 0, 0, 0)),
            pl.BlockSpec((1, 3, W, 64), lambda p, l: (p, 0, 0, 0)),
            pl.BlockSpec((1, 1, W), lambda p, l: (p, 0, 0)),
            pl.BlockSpec((1, 1, 16, W, W), lambda p, l: (p, l, 0, 0, 0)),
            pl.BlockSpec((1, 1, 8, W), lambda p, l: (p, l, 0, 0)),
            pl.BlockSpec((1, 3, _CODE_DIM, W), lambda p, l: (p, 0, 0, 0)),
            pl.BlockSpec((1, 1, _CODE_DIM), lambda p, l: (p, 0, 0)),
            pl.BlockSpec((1, _NB_CODE, _CODE_DIM), lambda p, l: (p, 0, 0)),
        ],
        out_specs=[
            pl.BlockSpec((1, _B, 8, _CODE_DIM), lambda p, l: (p, 0, 0, 0)),
            pl.BlockSpec((1, 8, 128), lambda p, l: (p, 0, 0)),
        ],
        out_shape=[
            jax.ShapeDtypeStruct((5, _B, 8, _CODE_DIM), f32),
            jax.ShapeDtypeStruct((5, 8, 128), f32),
        ],
        scratch_shapes=[pltpu.VMEM((_B, _T0, W), f32)],
    )(*enc_in)

    dec_in = [q5] + [jnp.stack(v) for v in
                     (dwin, dbin, dwlev, dblev, wmid, bmid, dwout, dbout)]
    y5 = pl.pallas_call(
        _dec_kernel,
        grid=grid,
        in_specs=[
            pl.BlockSpec((1, _B, 8, _CODE_DIM), lambda p, l: (p, 0, 0, 0)),
            pl.BlockSpec((1, 3, W, _CODE_DIM), lambda p, l: (p, 0, 0, 0)),
            pl.BlockSpec((1, 1, W), lambda p, l: (p, 0, 0)),
            pl.BlockSpec((1, 1, 15, W, W), lambda p, l: (p, l, 0, 0, 0)),
            pl.BlockSpec((1, 1, 8, W), lambda p, l: (p, l, 0, 0)),
            pl.BlockSpec((1, 3, W, W), lambda p, l: (p, 0, 0, 0)),
            pl.BlockSpec((1, 1, W), lambda p, l: (p, 0, 0)),
            pl.BlockSpec((1, 3, 64, W), lambda p, l: (p, 0, 0, 0)),
            pl.BlockSpec((1, 1, 64), lambda p, l: (p, 0, 0)),
        ],
        out_specs=pl.BlockSpec((1, _B, _T0, 64), lambda p, l: (p, 0, 0, 0)),
        out_shape=jax.ShapeDtypeStruct((5, _B, _T0, 64), f32),
        scratch_shapes=[pltpu.VMEM((_B, _T0, W), f32)],
    )(*dec_in)

    ys = [y5[i][:, :, :_PART_DIM[n]] for i, n in enumerate(_PARTS)]
    motion = _shift_upper_up(_merge(ys[0], ys[1], ys[2], ys[3], ys[4]))
    loss = jnp.sum(stats5[:, 0, 0])
    perplexity = stats5[4, 1, 0]
    return motion, loss, perplexity
